# baseline (speedup 1.0000x reference)
import functools

import jax
import jax.numpy as jnp
from jax import lax
from jax.experimental import pallas as pl
from jax.experimental.pallas import tpu as pltpu
from jax.experimental.pallas import tpu_sc as plsc

B = 16384
D = 64
L = 16
NC = 2
NS = 16
NW = NC * NS
BPW = B // NW
SEG = 128
NSEG = BPW // SEG
CHUNK = 256
NCHUNK = BPW // CHUNK
GPC = CHUNK // L


def _gmf_body(idx_u_hbm, idx_i_hbm, ut_hbm, it_hbm, w_hbm, b_hbm, out_hbm,
              idx_u_v, idx_i_v, lin_u_v, lin_i_v, ul_v, il_v, w_v, b_v,
              out_v, sem):
    wid = lax.axis_index("s") * NC + lax.axis_index("c")
    base = wid * BPW

    pltpu.sync_copy(idx_u_hbm.at[pl.ds(wid * NSEG, NSEG)], idx_u_v)
    pltpu.sync_copy(idx_i_hbm.at[pl.ds(wid * NSEG, NSEG)], idx_i_v)
    pltpu.sync_copy(w_hbm, w_v)
    pltpu.sync_copy(b_hbm, b_v)

    for j in range(NSEG):
        for c in range(SEG // L):
            lin_u_v[j, pl.ds(c * L, L)] = lax.shift_right_logical(
                idx_u_v[j, pl.ds(c * L, L)], 1)
            lin_i_v[j, pl.ds(c * L, L)] = lax.shift_right_logical(
                idx_i_v[j, pl.ds(c * L, L)], 1)

    bvec = b_v[...]
    iota = lax.iota(jnp.int32, L)

    for h in range(NCHUNK):
        copies = []
        for j in range(2):
            copies.append(pltpu.async_copy(
                ut_hbm.at[lin_u_v.at[2 * h + j]],
                ul_v.at[pl.ds(j * SEG, SEG)], sem))
            copies.append(pltpu.async_copy(
                it_hbm.at[lin_i_v.at[2 * h + j]],
                il_v.at[pl.ds(j * SEG, SEG)], sem))
        for c in copies:
            c.wait()

        def group(g, carry, h=h):
            j = 2 * h + (g >> 3)
            co = (g & 7) * L
            iv_u = idx_u_v[j, pl.ds(co, L)]
            iv_i = idx_i_v[j, pl.ds(co, L)]
            rows = g * L + iota
            col_u = (iv_u & 1) * D
            col_i = (iv_i & 1) * D

            def dstep(d, acc_cols):
                acc, cu, ci = acc_cols
                u = plsc.load_gather(ul_v, [rows, cu])
                i = plsc.load_gather(il_v, [rows, ci])
                w = plsc.load_gather(w_v, [jnp.full((L,), d, jnp.int32)])
                acc = acc + (u * i) * w
                return (acc, cu + 1, ci + 1)

            acc, _, _ = lax.fori_loop(
                0, D, dstep, (jnp.zeros((L,), jnp.float32), col_u, col_i),
                unroll=8)
            x = acc + bvec
            off = pl.multiple_of(h * CHUNK + g * L, L)
            out_v[pl.ds(off, L)] = 1.0 / (1.0 + jnp.exp(-x))
            return carry

        lax.fori_loop(0, GPC, group, 0)

    pltpu.sync_copy(out_v, out_hbm.at[pl.ds(base, BPW)])


_gmf_call = functools.partial(
    pl.kernel,
    mesh=plsc.VectorSubcoreMesh(core_axis_name="c", subcore_axis_name="s"),
    out_type=jax.ShapeDtypeStruct((B,), jnp.float32),
    compiler_params=pltpu.CompilerParams(
        needs_layout_passes=False, use_tc_tiling_on_sc=True),
    scratch_types=[
        pltpu.VMEM((NSEG, SEG), jnp.int32),
        pltpu.VMEM((NSEG, SEG), jnp.int32),
        pltpu.VMEM((NSEG, SEG), jnp.int32),
        pltpu.VMEM((NSEG, SEG), jnp.int32),
        pltpu.VMEM((CHUNK, 2 * D), jnp.float32),
        pltpu.VMEM((CHUNK, 2 * D), jnp.float32),
        pltpu.VMEM((D,), jnp.float32),
        pltpu.VMEM((L,), jnp.float32),
        pltpu.VMEM((BPW,), jnp.float32),
        pltpu.SemaphoreType.DMA,
    ],
)


def kernel(user_entries, item_entries, user_table, item_table, W, b):
    idx_u = user_entries.astype(jnp.int32).reshape(NW * NSEG, SEG)
    idx_i = item_entries.astype(jnp.int32).reshape(NW * NSEG, SEG)
    ut_lines = user_table.reshape(user_table.shape[0] // 2, 2 * D)
    it_lines = item_table.reshape(item_table.shape[0] // 2, 2 * D)
    w_flat = W.astype(jnp.float32).reshape(D)
    b16 = jnp.broadcast_to(b.astype(jnp.float32).reshape(()), (L,))
    return _gmf_call(_gmf_body)(idx_u, idx_i, ut_lines, it_lines, w_flat, b16)


# speedup vs baseline: 3.3341x; 3.3341x over previous
import functools

import jax
import jax.numpy as jnp
from jax import lax
from jax.experimental import pallas as pl
from jax.experimental.pallas import tpu as pltpu
from jax.experimental.pallas import tpu_sc as plsc

B = 16384
D = 64
L = 16
NC = 2
NS = 16
NW = NC * NS
BPW = B // NW
NSEG = 4
SEG = BPW // NSEG
NG = BPW // L

ROWS = 1000000
SUB = 8
GRAN = 16


def _gmf_body(idx_u_hbm, idx_i_hbm, ut_hbm, it_hbm, w_hbm, b_hbm, out_hbm,
              idx_u_v, idx_i_v, u4_v, i4_v, w_v, wspl_v, b_v, out_v, sem):
    wid = lax.axis_index("s") * NC + lax.axis_index("c")
    base = wid * BPW

    pltpu.sync_copy(idx_u_hbm.at[pl.ds(wid * NSEG, NSEG)], idx_u_v)
    pltpu.sync_copy(idx_i_hbm.at[pl.ds(wid * NSEG, NSEG)], idx_i_v)
    pltpu.sync_copy(w_hbm, w_v)
    pltpu.sync_copy(b_hbm, b_v)

    for d in range(D):
        wspl_v[pl.ds(d * L, L)] = plsc.load_gather(
            w_v, [jnp.full((L,), d, jnp.int32)])

    bvec = b_v[...]
    iota = lax.iota(jnp.int32, L)
    lane_masks = [iota == r for r in range(L)]
    zero = jnp.zeros((L,), jnp.int32)
    iota16 = iota * GRAN

    def group(g, carry):
        j = g >> 3
        co = (g & 7) * L
        iv_u = idx_u_v[j, pl.ds(co, L)]
        iv_i = idx_i_v[j, pl.ds(co, L)]
        le_u = iv_u & 15
        le_i = iv_i & 15
        copies = []
        for r in range(L):
            su = jnp.max(jnp.where(lane_masks[r], iv_u, zero))
            si = jnp.max(jnp.where(lane_masks[r], iv_i, zero))
            ou = pl.multiple_of(su & -16, 128)
            oi = pl.multiple_of(si & -16, 128)
            for k in range(SUB):
                copies.append(pltpu.async_copy(
                    ut_hbm.at[k, :, pl.ds(ou, GRAN)],
                    u4_v.at[pl.ds(k * SUB, SUB), pl.ds(r * GRAN, GRAN)], sem))
                copies.append(pltpu.async_copy(
                    it_hbm.at[k, :, pl.ds(oi, GRAN)],
                    i4_v.at[pl.ds(k * SUB, SUB), pl.ds(r * GRAN, GRAN)], sem))
        for cp in copies:
            cp.wait()

        cols_u = iota16 + le_u
        cols_i = iota16 + le_i
        acc = jnp.zeros((L,), jnp.float32)
        for d in range(D):
            rows = jnp.full((L,), d, jnp.int32)
            u = plsc.load_gather(u4_v, [rows, cols_u])
            i = plsc.load_gather(i4_v, [rows, cols_i])
            acc = acc + (u * i) * wspl_v[pl.ds(d * L, L)]
        x = acc + bvec
        off = pl.multiple_of(g * L, L)
        out_v[pl.ds(off, L)] = 1.0 / (1.0 + jnp.exp(-x))
        return carry

    lax.fori_loop(0, NG, group, 0)

    pltpu.sync_copy(out_v, out_hbm.at[pl.ds(base, BPW)])


_gmf_call = functools.partial(
    pl.kernel,
    mesh=plsc.VectorSubcoreMesh(core_axis_name="c", subcore_axis_name="s"),
    out_type=jax.ShapeDtypeStruct((B,), jnp.float32),
    compiler_params=pltpu.CompilerParams(
        needs_layout_passes=False, use_tc_tiling_on_sc=True),
    scratch_types=[
        pltpu.VMEM((NSEG, SEG), jnp.int32),
        pltpu.VMEM((NSEG, SEG), jnp.int32),
        pltpu.VMEM((D, L * GRAN), jnp.float32),
        pltpu.VMEM((D, L * GRAN), jnp.float32),
        pltpu.VMEM((D,), jnp.float32),
        pltpu.VMEM((D * L,), jnp.float32),
        pltpu.VMEM((L,), jnp.float32),
        pltpu.VMEM((BPW,), jnp.float32),
        pltpu.SemaphoreType.DMA,
    ],
)


def kernel(user_entries, item_entries, user_table, item_table, W, b):
    idx_u = user_entries.astype(jnp.int32).reshape(NW * NSEG, SEG)
    idx_i = item_entries.astype(jnp.int32).reshape(NW * NSEG, SEG)
    ut3 = user_table.T.reshape(SUB, SUB, ROWS)
    it3 = item_table.T.reshape(SUB, SUB, ROWS)
    w_flat = W.astype(jnp.float32).reshape(D)
    b16 = jnp.broadcast_to(b.astype(jnp.float32).reshape(()), (L,))
    return _gmf_call(_gmf_body)(idx_u, idx_i, ut3, it3, w_flat, b16)


# speedup vs baseline: 6.6775x; 2.0028x over previous
import functools

import jax
import jax.numpy as jnp
from jax import lax
from jax.experimental import pallas as pl
from jax.experimental.pallas import tpu as pltpu
from jax.experimental.pallas import tpu_sc as plsc

B = 16384
D = 64
L = 16
NC = 2
NS = 16
NW = NC * NS
BPW = B // NW
NSEG = 4
SEG = BPW // NSEG
NG = BPW // L

ROWS = 1000000
SUB = 8
GRAN = 16


def _gmf_body(idx_u_hbm, idx_i_hbm, ut_hbm, it_hbm, w_hbm, b_hbm, out_hbm,
              idx_u_v, idx_i_v, u4_v, i4_v, w_v, wspl_v, b_v, out_v, sem):
    wid = lax.axis_index("s") * NC + lax.axis_index("c")
    base = wid * BPW

    pltpu.sync_copy(idx_u_hbm.at[pl.ds(wid * NSEG, NSEG)], idx_u_v)
    pltpu.sync_copy(idx_i_hbm.at[pl.ds(wid * NSEG, NSEG)], idx_i_v)
    pltpu.sync_copy(w_hbm, w_v)
    pltpu.sync_copy(b_hbm, b_v)

    for d in range(D):
        wspl_v[pl.ds(d * L, L)] = plsc.load_gather(
            w_v, [jnp.full((L,), d, jnp.int32)])

    bvec = b_v[...]
    iota = lax.iota(jnp.int32, L)
    lane_masks = [iota == r for r in range(L)]
    zero = jnp.zeros((L,), jnp.int32)
    iota16 = iota * GRAN

    def group(g, carry):
        j = g >> 3
        co = (g & 7) * L
        iv_u = idx_u_v[j, pl.ds(co, L)]
        iv_i = idx_i_v[j, pl.ds(co, L)]
        le_u = iv_u & 15
        le_i = iv_i & 15
        copies = []
        for r in range(L):
            su = jnp.max(jnp.where(lane_masks[r], iv_u, zero))
            si = jnp.max(jnp.where(lane_masks[r], iv_i, zero))
            ou = pl.multiple_of(su & -16, 128)
            oi = pl.multiple_of(si & -16, 128)
            copies.append(pltpu.async_copy(
                ut_hbm.at[:, :, pl.ds(ou, GRAN)],
                u4_v.at[:, :, pl.ds(r * GRAN, GRAN)], sem))
            copies.append(pltpu.async_copy(
                it_hbm.at[:, :, pl.ds(oi, GRAN)],
                i4_v.at[:, :, pl.ds(r * GRAN, GRAN)], sem))
        for cp in copies:
            cp.wait()

        cols_u = iota16 + le_u
        cols_i = iota16 + le_i
        acc = jnp.zeros((L,), jnp.float32)
        for d in range(D):
            kk = jnp.full((L,), d // SUB, jnp.int32)
            ss = jnp.full((L,), d % SUB, jnp.int32)
            u = plsc.load_gather(u4_v, [kk, ss, cols_u])
            i = plsc.load_gather(i4_v, [kk, ss, cols_i])
            acc = acc + (u * i) * wspl_v[pl.ds(d * L, L)]
        x = acc + bvec
        off = pl.multiple_of(g * L, L)
        out_v[pl.ds(off, L)] = 1.0 / (1.0 + jnp.exp(-x))
        return carry

    lax.fori_loop(0, NG, group, 0)

    pltpu.sync_copy(out_v, out_hbm.at[pl.ds(base, BPW)])


_gmf_call = functools.partial(
    pl.kernel,
    mesh=plsc.VectorSubcoreMesh(core_axis_name="c", subcore_axis_name="s"),
    out_type=jax.ShapeDtypeStruct((B,), jnp.float32),
    compiler_params=pltpu.CompilerParams(
        needs_layout_passes=False, use_tc_tiling_on_sc=True),
    scratch_types=[
        pltpu.VMEM((NSEG, SEG), jnp.int32),
        pltpu.VMEM((NSEG, SEG), jnp.int32),
        pltpu.VMEM((SUB, SUB, L * GRAN), jnp.float32),
        pltpu.VMEM((SUB, SUB, L * GRAN), jnp.float32),
        pltpu.VMEM((D,), jnp.float32),
        pltpu.VMEM((D * L,), jnp.float32),
        pltpu.VMEM((L,), jnp.float32),
        pltpu.VMEM((BPW,), jnp.float32),
        pltpu.SemaphoreType.DMA,
    ],
)


def kernel(user_entries, item_entries, user_table, item_table, W, b):
    idx_u = user_entries.astype(jnp.int32).reshape(NW * NSEG, SEG)
    idx_i = item_entries.astype(jnp.int32).reshape(NW * NSEG, SEG)
    ut3 = user_table.T.reshape(SUB, SUB, ROWS)
    it3 = item_table.T.reshape(SUB, SUB, ROWS)
    w_flat = W.astype(jnp.float32).reshape(D)
    b16 = jnp.broadcast_to(b.astype(jnp.float32).reshape(()), (L,))
    return _gmf_call(_gmf_body)(idx_u, idx_i, ut3, it3, w_flat, b16)
